# Initial kernel scaffold; baseline (speedup 1.0000x reference)
import jax
import jax.numpy as jnp
from jax.experimental import pallas as pl


def kernel(x, edge_index, W1, b1, W2, b2):
    raise NotImplementedError("write your pallas kernel here")


# speedup vs baseline: 3.6497x; 3.6497x over previous
import functools

import jax
import jax.numpy as jnp
from jax import lax
from jax.experimental import pallas as pl
from jax.experimental.pallas import tpu as pltpu
from jax.experimental.pallas import tpu_sc as plsc

N_NODES = 10000
D = 256
NC = 2
NS = 16
NW = NC * NS

NPAD = 10240
ROWS_PER_TILE = NPAD // NW
NHIST = 10368
CH = 128
SCAN = 4096
CHD = 64
BM = 512

def _sc_mesh():
    return plsc.VectorSubcoreMesh(
        core_axis_name="c", subcore_axis_name="s",
        num_cores=NC, num_subcores=NS)


def _hist_body(ept_a, dst_hbm, out_hbm, hist, dstbuf):
    cid = lax.axis_index("c")
    sid = lax.axis_index("s")
    wid = cid * NS + sid
    pltpu.sync_copy(dst_hbm.at[pl.ds(wid * ept_a, ept_a)], dstbuf)
    zeros = jnp.zeros((16,), jnp.float32)

    def zbody(i, c):
        hist[pl.ds(i * 16, 16)] = zeros
        return c
    lax.fori_loop(0, NHIST // 16, zbody, 0)

    ones = jnp.ones((16,), jnp.float32)

    def body(i, c):
        dv = dstbuf[pl.ds(i * 16, 16)]
        plsc.addupdate_scatter(hist, [dv], ones)
        return c
    lax.fori_loop(0, ept_a // 16, body, 0)
    pltpu.sync_copy(hist, out_hbm.at[wid])


def _propagate_body(epad, z_hbm, src_hbm, dst_hbm, acc_hbm,
                    acc, gbuf, srcc, dstc, wl_src, wl_dst):
    cid = lax.axis_index("c")
    sid = lax.axis_index("s")
    wid = cid * NS + sid
    row_lo = wid * ROWS_PER_TILE

    pltpu.sync_copy(z_hbm.at[pl.ds(row_lo, ROWS_PER_TILE)],
                    acc.at[pl.ds(0, ROWS_PER_TILE)])

    zi = jnp.zeros((16,), jnp.int32)

    def zb(i, c):
        wl_src[pl.ds(i * 16, 16)] = zi
        wl_dst[pl.ds(i * 16, 16)] = zi
        return c
    lax.fori_loop(0, (SCAN + CHD) // 16, zb, 0)

    def accumulate_batch(b):
        for rb in range(CHD // 16):
            locv = wl_dst[pl.ds(b * CHD + rb * 16, 16)]
            for l in range(16):
                loc = locv[l]
                r = rb * 16 + l
                for cc in range(D // 16):
                    plsc.addupdate(acc.at[loc, pl.ds(cc * 16, 16)],
                                   gbuf[r, pl.ds(cc * 16, 16)])

    def drain_batches(nb, wcount):
        def dbody(b, c):
            pltpu.sync_copy(z_hbm.at[wl_src.at[pl.ds(b * CHD, CHD)]], gbuf)
            accumulate_batch(b)
            return c
        lax.fori_loop(0, nb, dbody, 0)
        base = nb * CHD

        def mv(k, c):
            sv = wl_src[pl.ds(base + k * 16, 16)]
            dv = wl_dst[pl.ds(base + k * 16, 16)]
            wl_src[pl.ds(k * 16, 16)] = sv
            wl_dst[pl.ds(k * 16, 16)] = dv
            return c
        lax.fori_loop(0, CHD // 16, mv, 0)
        return wcount - nb * CHD

    def sbody(sc_i, wcount):
        off = sc_i * SCAN
        pltpu.sync_copy(src_hbm.at[pl.ds(off, SCAN)], srcc)
        pltpu.sync_copy(dst_hbm.at[pl.ds(off, SCAN)], dstc)

        def vbody(v, wc):
            d = dstc[pl.ds(v * 16, 16)]
            s = srcc[pl.ds(v * 16, 16)]
            mask = (d >= row_lo) & (d < row_lo + ROWS_PER_TILE)
            plsc.store_compressed(wl_src.at[pl.ds(wc, 16)], s, mask=mask)
            plsc.store_compressed(wl_dst.at[pl.ds(wc, 16)], d - row_lo, mask=mask)
            return wc + jnp.sum(mask.astype(jnp.int32))
        wcount = lax.fori_loop(0, SCAN // 16, vbody, wcount)
        return drain_batches(wcount // CHD, wcount)
    wcount = lax.fori_loop(0, epad // SCAN, sbody, 0)

    @pl.when(wcount > 0)
    def _():
        dummy = jnp.full((16,), ROWS_PER_TILE, jnp.int32)
        for k in range(CHD // 16):
            wl_dst[pl.ds(wcount + k * 16, 16)] = dummy
        pltpu.sync_copy(z_hbm.at[wl_src.at[pl.ds(0, CHD)]], gbuf)
        accumulate_batch(0)

    pltpu.sync_copy(acc.at[pl.ds(0, ROWS_PER_TILE)],
                    acc_hbm.at[pl.ds(row_lo, ROWS_PER_TILE)])


def _hist_call(dst_p, epad):
    ept_a = epad // NW
    return pl.kernel(
        functools.partial(_hist_body, ept_a),
        out_type=jax.ShapeDtypeStruct((NW, NHIST), jnp.float32),
        mesh=_sc_mesh(),
        compiler_params=pltpu.CompilerParams(needs_layout_passes=False),
        scratch_types=[
            pltpu.VMEM((NHIST,), jnp.float32),
            pltpu.VMEM((ept_a,), jnp.int32),
        ],
    )(dst_p)


def _propagate_call(z, src_p, dst_p, epad):
    return pl.kernel(
        functools.partial(_propagate_body, epad),
        out_type=jax.ShapeDtypeStruct((NPAD, D), jnp.float32),
        mesh=_sc_mesh(),
        compiler_params=pltpu.CompilerParams(needs_layout_passes=False),
        scratch_types=[
            pltpu.VMEM((ROWS_PER_TILE + 1, D), jnp.float32),
            pltpu.VMEM((CHD, D), jnp.float32),
            pltpu.VMEM((SCAN,), jnp.int32),
            pltpu.VMEM((SCAN,), jnp.int32),
            pltpu.VMEM((SCAN + CHD + 16,), jnp.int32),
            pltpu.VMEM((SCAN + CHD + 16,), jnp.int32),
        ],
    )(z, src_p, dst_p)


def _dinv_body(hist_ref, o_ref):
    deg = jnp.sum(hist_ref[...], axis=0, keepdims=True) + 1.0
    o_ref[...] = lax.rsqrt(deg)


def _mm_scale_body(x_ref, w_ref, dinv_ref, o_ref):
    o_ref[...] = jnp.dot(x_ref[...], w_ref[...],
                         preferred_element_type=jnp.float32) * dinv_ref[...]


def _layer2_body(acc_ref, dinv_ref, b_ref, w_ref, o_ref):
    h = jnp.maximum(acc_ref[...] * dinv_ref[...] + b_ref[...], 0.0)
    o_ref[...] = jnp.dot(h, w_ref[...],
                         preferred_element_type=jnp.float32) * dinv_ref[...]


def _final_body(acc_ref, dinv_ref, b_ref, o_ref):
    o_ref[...] = acc_ref[...] * dinv_ref[...] + b_ref[...]


def kernel(x, edge_index, W1, b1, W2, b2):
    n, d = x.shape
    e = edge_index.shape[1]
    assert n == N_NODES and d == D
    epad = -(-e // SCAN) * SCAN

    ei = edge_index.astype(jnp.int32)
    src_p = jnp.concatenate([ei[0], jnp.zeros((epad - e,), jnp.int32)])
    dst_p = jnp.concatenate(
        [ei[1], jnp.full((epad - e,), NPAD, jnp.int32)])
    x_p = jnp.concatenate(
        [x, jnp.zeros((NPAD - n, d), jnp.float32)], axis=0)
    b1r = b1.reshape(1, d)
    b2r = b2.reshape(1, d)

    hist = _hist_call(dst_p, epad)

    dinv_row = pl.pallas_call(
        _dinv_body,
        in_specs=[pl.BlockSpec((NW, NHIST), lambda: (0, 0))],
        out_specs=pl.BlockSpec((1, NHIST), lambda: (0, 0)),
        out_shape=jax.ShapeDtypeStruct((1, NHIST), jnp.float32),
    )(hist)
    dinv_col = dinv_row[0, :NPAD].reshape(NPAD, 1)

    grid = (NPAD // BM,)
    z1 = pl.pallas_call(
        _mm_scale_body,
        grid=grid,
        in_specs=[pl.BlockSpec((BM, d), lambda i: (i, 0)),
                  pl.BlockSpec((d, d), lambda i: (0, 0)),
                  pl.BlockSpec((BM, 1), lambda i: (i, 0))],
        out_specs=pl.BlockSpec((BM, d), lambda i: (i, 0)),
        out_shape=jax.ShapeDtypeStruct((NPAD, d), jnp.float32),
    )(x_p, W1, dinv_col)

    acc1 = _propagate_call(z1, src_p, dst_p, epad)

    z2 = pl.pallas_call(
        _layer2_body,
        grid=grid,
        in_specs=[pl.BlockSpec((BM, d), lambda i: (i, 0)),
                  pl.BlockSpec((BM, 1), lambda i: (i, 0)),
                  pl.BlockSpec((1, d), lambda i: (0, 0)),
                  pl.BlockSpec((d, d), lambda i: (0, 0))],
        out_specs=pl.BlockSpec((BM, d), lambda i: (i, 0)),
        out_shape=jax.ShapeDtypeStruct((NPAD, d), jnp.float32),
    )(acc1, dinv_col, b1r, W2)

    acc2 = _propagate_call(z2, src_p, dst_p, epad)

    out = pl.pallas_call(
        _final_body,
        grid=grid,
        in_specs=[pl.BlockSpec((BM, d), lambda i: (i, 0)),
                  pl.BlockSpec((BM, 1), lambda i: (i, 0)),
                  pl.BlockSpec((1, d), lambda i: (0, 0))],
        out_specs=pl.BlockSpec((BM, d), lambda i: (i, 0)),
        out_shape=jax.ShapeDtypeStruct((NPAD, d), jnp.float32),
    )(acc2, dinv_col, b2r)
    return out[:n]
